# baseline (speedup 1.0000x reference)
import functools

import jax
import jax.numpy as jnp
from jax import lax
from jax.experimental import pallas as pl
from jax.experimental.pallas import tpu as pltpu
from jax.experimental.pallas import tpu_sc as plsc

GAMMA = 12.0

NUM_ENTS = 1000000
DIM = 64
W = DIM // 2
BATCH = 4096
NEG = 256

NUM_WORKERS = 32
BPW = BATCH // NUM_WORKERS
CHUNK = 2
NCHUNKS = BPW // CHUNK
RPC = CHUNK * NEG
NGATH = RPC // 128


def _sc_body(x_hbm, tgt_hbm, pos_hbm, neg_hbm, ent_hbm,
             pos_out, neg_out,
             tgt_v, posidx_v, negidx0, negidx1, pred_v, predw_v, posrow_v,
             negrow0, negrow1, posout_v, negout0, negout1,
             isem0, isem1, rsem0, rsem1, osem0, osem1):
  nc = 2
  wid = lax.axis_index("s") * nc + lax.axis_index("c")
  base = wid * BPW
  iota = lax.iota(jnp.int32, 16)

  negidx = (negidx0, negidx1)
  negrow = (negrow0, negrow1)
  negout = (negout0, negout1)
  isem = (isem0, isem1)
  rsem = (rsem0, rsem1)
  osem = (osem0, osem1)

  def copy_idx(c, buf, sem=None):
    for k in range(CHUNK):
      src = neg_hbm.at[base + c * CHUNK + k]
      dst = negidx[buf].at[pl.ds(k * NEG, NEG)]
      if sem is None:
        pltpu.sync_copy(src, dst)
      else:
        pltpu.async_copy(src, dst, sem)

  def wait_idx(c, buf, sem):
    for k in range(CHUNK):
      pltpu.make_async_copy(neg_hbm.at[base + c * CHUNK + k],
                            negidx[buf].at[pl.ds(k * NEG, NEG)], sem).wait()

  def out_slice(c):
    return neg_out.at[pl.ds((base + c * CHUNK) * NEG, RPC)]

  pltpu.sync_copy(tgt_hbm.at[pl.ds(base, BPW)], tgt_v)
  pltpu.sync_copy(pos_hbm.at[pl.ds(base, BPW)], posidx_v)
  pltpu.sync_copy(x_hbm.at[tgt_v], pred_v)
  pltpu.sync_copy(ent_hbm.at[posidx_v], posrow_v)

  def start_rows(buf):
    for j in range(NGATH):
      pltpu.async_copy(
          ent_hbm.at[negidx[buf].at[pl.ds(j * 128, 128)]],
          negrow[buf].at[pl.ds(j * 128, 128), :],
          rsem[buf])

  def wait_rows(buf):
    for j in range(NGATH):
      pltpu.make_async_copy(
          ent_hbm.at[negidx[buf].at[pl.ds(j * 128, 128)]],
          negrow[buf].at[pl.ds(j * 128, 128), :],
          rsem[buf]).wait()

  copy_idx(0, 0)
  start_rows(0)
  copy_idx(1, 1, isem[1])

  def pack_row(r, carry):
    src = pred_v.at[r]
    for h in range(2):
      ev = plsc.load_gather(src, [h * W + iota * 2])
      od = plsc.load_gather(src, [h * W + iota * 2 + 1])
      packed = plsc.bitcast(
          plsc.pack(ev, od, format=plsc.PackFormat.INTERLEAVED), jnp.int32)
      predw_v[r, pl.ds(h * 16, 16)] = packed
    return carry

  lax.fori_loop(0, BPW, pack_row, 0)

  def l1_terms(word_i32, prot_i32):
    ebf = plsc.bitcast(word_i32, jnp.bfloat16)
    pbf = plsc.bitcast(prot_i32, jnp.bfloat16)
    a = jnp.abs(ebf - pbf)
    lo, hi = plsc.unpack(a, format=plsc.PackFormat.INTERLEAVED)
    return lo.astype(jnp.float32) + hi.astype(jnp.float32)

  def pos_group(g, carry):
    possub = posrow_v.at[pl.ds(g * 16, 16), :]
    predsub = predw_v.at[pl.ds(g * 16, 16), :]

    def w_body(w0, acc, possub=possub, predsub=predsub):
      cols = jnp.bitwise_and(iota + w0, W - 1)
      ev = plsc.load_gather(possub, [iota, cols])
      pv = plsc.load_gather(predsub, [iota, cols])
      return acc + l1_terms(ev, pv)

    acc = lax.fori_loop(0, W, w_body, jnp.zeros((16,), jnp.float32),
                        unroll=8)
    posout_v[pl.ds(g * 16, 16)] = GAMMA - acc
    return carry

  lax.fori_loop(0, BPW // 16, pos_group, 0)
  pltpu.sync_copy(posout_v, pos_out.at[pl.ds(base, BPW)])

  def compute_chunk(c, buf):
    for bb in range(CHUNK):
      prow = c * CHUNK + bb
      pred_row = predw_v.at[prow]

      def w_body(w0, accs, pred_row=pred_row, bb=bb, buf=buf):
        cols = jnp.bitwise_and(iota + w0, W - 1)
        prot = plsc.load_gather(pred_row, [cols])
        pbf = plsc.bitcast(prot, jnp.bfloat16)
        new = []
        for g in range(16):
          sub = negrow[buf].at[pl.ds(bb * NEG + g * 16, 16), :]
          ev = plsc.load_gather(sub, [iota, cols])
          ebf = plsc.bitcast(ev, jnp.bfloat16)
          a = jnp.abs(ebf - pbf)
          lo, hi = plsc.unpack(a, format=plsc.PackFormat.INTERLEAVED)
          new.append(accs[g] + (lo.astype(jnp.float32) +
                                hi.astype(jnp.float32)))
        return tuple(new)

      accs = lax.fori_loop(0, W, w_body,
                           (jnp.zeros((16,), jnp.float32),) * 16,
                           unroll=4)
      for g in range(16):
        negout[buf][pl.ds(bb * NEG + g * 16, 16)] = GAMMA - accs[g]

  def handle(c, buf):
    wait_rows(buf)

    nbuf = 1 - buf

    @pl.when(c + 1 < NCHUNKS)
    def _():
      wait_idx(c + 1, nbuf, isem[nbuf])
      start_rows(nbuf)

    @pl.when(c + 2 < NCHUNKS)
    def _():
      copy_idx(c + 2, buf, isem[buf])

    @pl.when(c >= 2)
    def _():
      pltpu.make_async_copy(negout[buf], out_slice(c - 2), osem[buf]).wait()

    compute_chunk(c, buf)
    pltpu.async_copy(negout[buf], out_slice(c), osem[buf])

  def pair_body(p, carry):
    handle(2 * p, 0)
    handle(2 * p + 1, 1)
    return carry

  lax.fori_loop(0, NCHUNKS // 2, pair_body, 0)

  pltpu.make_async_copy(negout[0], out_slice(NCHUNKS - 2), osem[0]).wait()
  pltpu.make_async_copy(negout[1], out_slice(NCHUNKS - 1), osem[1]).wait()


@jax.jit
def _sc_kernel(x, tgt, pos, neg, ent_w):
  mesh = plsc.VectorSubcoreMesh(core_axis_name="c", subcore_axis_name="s")
  f = functools.partial(
      pl.kernel,
      mesh=mesh,
      compiler_params=pltpu.CompilerParams(
          needs_layout_passes=False, use_tc_tiling_on_sc=False),
      out_type=(
          jax.ShapeDtypeStruct((BATCH,), jnp.float32),
          jax.ShapeDtypeStruct((BATCH * NEG,), jnp.float32),
      ),
      scratch_types=[
          pltpu.VMEM((BPW,), jnp.int32),
          pltpu.VMEM((BPW,), jnp.int32),
          pltpu.VMEM((RPC,), jnp.int32),
          pltpu.VMEM((RPC,), jnp.int32),
          pltpu.VMEM((BPW, DIM), jnp.float32),
          pltpu.VMEM((BPW, W), jnp.int32),
          pltpu.VMEM((BPW, W), jnp.int32),
          pltpu.VMEM((RPC, W), jnp.int32),
          pltpu.VMEM((RPC, W), jnp.int32),
          pltpu.VMEM((BPW,), jnp.float32),
          pltpu.VMEM((RPC,), jnp.float32),
          pltpu.VMEM((RPC,), jnp.float32),
          pltpu.SemaphoreType.DMA,
          pltpu.SemaphoreType.DMA,
          pltpu.SemaphoreType.DMA,
          pltpu.SemaphoreType.DMA,
          pltpu.SemaphoreType.DMA,
          pltpu.SemaphoreType.DMA,
      ],
  )(_sc_body)
  return f(x, tgt, pos, neg, ent_w)


def kernel(x, target_node_idxes, positive_samples, negative_samples,
           ent_embedding):
  tgt = target_node_idxes.astype(jnp.int32)
  pos = positive_samples.astype(jnp.int32)
  neg = negative_samples.astype(jnp.int32)
  ent_w = jax.lax.bitcast_convert_type(
      ent_embedding.astype(jnp.bfloat16).reshape(NUM_ENTS, W, 2),
      jnp.int32)
  pos_l, neg_l = _sc_kernel(x, tgt, pos, neg, ent_w)
  return pos_l[:, None], neg_l.reshape(BATCH, NEG)


# speedup vs baseline: 2.4822x; 2.4822x over previous
import functools

import jax
import jax.numpy as jnp
from jax import lax
from jax.experimental import pallas as pl
from jax.experimental.pallas import tpu as pltpu
from jax.experimental.pallas import tpu_sc as plsc

GAMMA = 12.0

NUM_ENTS = 1000000
DIM = 64
BATCH = 4096
NEG = 256

NUM_WORKERS = 32
BPW = BATCH // NUM_WORKERS
CHUNK = 2
NCHUNKS = BPW // CHUNK
RPC = CHUNK * NEG
NGATH = RPC // 128


def _sc_body(x_hbm, tgt_hbm, pos_hbm, neg_hbm, ent_hbm,
             pos_out, neg_out,
             tgt_v, posidx_v, negidx0, negidx1, pred_v, posrow_v,
             negrow0, negrow1, posout_v, negout0, negout1,
             isem0, isem1, rsem0, rsem1, osem0, osem1):
  nc = 2
  wid = lax.axis_index("s") * nc + lax.axis_index("c")
  base = wid * BPW
  iota = lax.iota(jnp.int32, 16)

  negidx = (negidx0, negidx1)
  negrow = (negrow0, negrow1)
  negout = (negout0, negout1)
  isem = (isem0, isem1)
  rsem = (rsem0, rsem1)
  osem = (osem0, osem1)

  def copy_idx(c, buf, sem=None):
    for k in range(CHUNK):
      src = neg_hbm.at[base + c * CHUNK + k]
      dst = negidx[buf].at[pl.ds(k * NEG, NEG)]
      if sem is None:
        pltpu.sync_copy(src, dst)
      else:
        pltpu.async_copy(src, dst, sem)

  def wait_idx(c, buf, sem):
    for k in range(CHUNK):
      pltpu.make_async_copy(neg_hbm.at[base + c * CHUNK + k],
                            negidx[buf].at[pl.ds(k * NEG, NEG)], sem).wait()

  def out_slice(c):
    return neg_out.at[pl.ds((base + c * CHUNK) * NEG, RPC)]

  pltpu.sync_copy(tgt_hbm.at[pl.ds(base, BPW)], tgt_v)
  pltpu.sync_copy(pos_hbm.at[pl.ds(base, BPW)], posidx_v)
  pltpu.sync_copy(x_hbm.at[tgt_v], pred_v)
  pltpu.sync_copy(ent_hbm.at[posidx_v], posrow_v)

  copy_idx(0, 0)
  for j in range(NGATH):
    pltpu.async_copy(
        ent_hbm.at[negidx0.at[pl.ds(j * 128, 128)]],
        negrow0.at[pl.ds(j * 128, 128), :],
        rsem0)
  copy_idx(1, 1, isem[1])

  def pos_group(g, carry):
    possub = posrow_v.at[pl.ds(g * 16, 16), :]
    predsub = pred_v.at[pl.ds(g * 16, 16), :]

    def d0_body(d0, acc, possub=possub, predsub=predsub):
      cols = jnp.bitwise_and(iota + d0, DIM - 1)
      ev = plsc.load_gather(possub, [iota, cols])
      pv = plsc.load_gather(predsub, [iota, cols])
      return acc + jnp.abs(ev - pv)

    acc = lax.fori_loop(0, DIM, d0_body, jnp.zeros((16,), jnp.float32),
                        unroll=8)
    posout_v[pl.ds(g * 16, 16)] = GAMMA - acc
    return carry

  lax.fori_loop(0, BPW // 16, pos_group, 0)
  pltpu.sync_copy(posout_v, pos_out.at[pl.ds(base, BPW)])

  def start_rows(buf):
    for j in range(NGATH):
      pltpu.async_copy(
          ent_hbm.at[negidx[buf].at[pl.ds(j * 128, 128)]],
          negrow[buf].at[pl.ds(j * 128, 128), :],
          rsem[buf])

  def wait_rows(buf):
    for j in range(NGATH):
      pltpu.make_async_copy(
          ent_hbm.at[negidx[buf].at[pl.ds(j * 128, 128)]],
          negrow[buf].at[pl.ds(j * 128, 128), :],
          rsem[buf]).wait()


  def compute_chunk(c, buf):
    for bb in range(CHUNK):
      prow = c * CHUNK + bb
      pred_row = pred_v.at[prow]

      def d0_body(d0, accs, pred_row=pred_row, bb=bb, buf=buf):
        cols = jnp.bitwise_and(iota + d0, DIM - 1)
        prot = plsc.load_gather(pred_row, [cols])
        new = []
        for g in range(16):
          sub = negrow[buf].at[pl.ds(bb * NEG + g * 16, 16), :]
          ev = plsc.load_gather(sub, [iota, cols])
          new.append(accs[g] + jnp.abs(ev - prot))
        return tuple(new)

      accs = lax.fori_loop(0, DIM, d0_body,
                           (jnp.zeros((16,), jnp.float32),) * 16,
                           unroll=4)
      for g in range(16):
        negout[buf][pl.ds(bb * NEG + g * 16, 16)] = GAMMA - accs[g]

  def handle(c, buf):
    wait_rows(buf)

    nbuf = 1 - buf

    @pl.when(c + 1 < NCHUNKS)
    def _():
      wait_idx(c + 1, nbuf, isem[nbuf])
      start_rows(nbuf)

    @pl.when(c + 2 < NCHUNKS)
    def _():
      copy_idx(c + 2, buf, isem[buf])

    @pl.when(c >= 2)
    def _():
      pltpu.make_async_copy(negout[buf], out_slice(c - 2), osem[buf]).wait()

    compute_chunk(c, buf)
    pltpu.async_copy(negout[buf], out_slice(c), osem[buf])

  def pair_body(p, carry):
    handle(2 * p, 0)
    handle(2 * p + 1, 1)
    return carry

  lax.fori_loop(0, NCHUNKS // 2, pair_body, 0)

  pltpu.make_async_copy(negout[0], out_slice(NCHUNKS - 2), osem[0]).wait()
  pltpu.make_async_copy(negout[1], out_slice(NCHUNKS - 1), osem[1]).wait()


@jax.jit
def _sc_kernel(x, tgt, pos, neg_flat, ent):
  mesh = plsc.VectorSubcoreMesh(core_axis_name="c", subcore_axis_name="s")
  f = functools.partial(
      pl.kernel,
      mesh=mesh,
      compiler_params=pltpu.CompilerParams(
          needs_layout_passes=False, use_tc_tiling_on_sc=False),
      out_type=(
          jax.ShapeDtypeStruct((BATCH,), jnp.float32),
          jax.ShapeDtypeStruct((BATCH * NEG,), jnp.float32),
      ),
      scratch_types=[
          pltpu.VMEM((BPW,), jnp.int32),
          pltpu.VMEM((BPW,), jnp.int32),
          pltpu.VMEM((RPC,), jnp.int32),
          pltpu.VMEM((RPC,), jnp.int32),
          pltpu.VMEM((BPW, DIM), jnp.float32),
          pltpu.VMEM((BPW, DIM), jnp.float32),
          pltpu.VMEM((RPC, DIM), jnp.float32),
          pltpu.VMEM((RPC, DIM), jnp.float32),
          pltpu.VMEM((BPW,), jnp.float32),
          pltpu.VMEM((RPC,), jnp.float32),
          pltpu.VMEM((RPC,), jnp.float32),
          pltpu.SemaphoreType.DMA,
          pltpu.SemaphoreType.DMA,
          pltpu.SemaphoreType.DMA,
          pltpu.SemaphoreType.DMA,
          pltpu.SemaphoreType.DMA,
          pltpu.SemaphoreType.DMA,
      ],
  )(_sc_body)
  return f(x, tgt, pos, neg_flat, ent)


def kernel(x, target_node_idxes, positive_samples, negative_samples,
           ent_embedding):
  tgt = target_node_idxes.astype(jnp.int32)
  pos = positive_samples.astype(jnp.int32)
  neg = negative_samples.astype(jnp.int32)
  pos_l, neg_l = _sc_kernel(x, tgt, pos, neg, ent_embedding)
  return pos_l[:, None], neg_l.reshape(BATCH, NEG)
